# Initial kernel scaffold; baseline (speedup 1.0000x reference)
import jax
import jax.numpy as jnp
from jax.experimental import pallas as pl


def kernel(user_factors, item_factors, test_items):
    raise NotImplementedError("write your pallas kernel here")


# speedup vs baseline: 1.8136x; 1.8136x over previous
import functools

import jax
import jax.numpy as jnp
from jax import lax
from jax.experimental import pallas as pl
from jax.experimental.pallas import tpu as pltpu
from jax.experimental.pallas import tpu_sc as plsc

U = 16384
N_ITEMS = 1000000
D = 64
L = 200

NW = 32
UPW = U // NW
CH = 64
IDXC = 100
NG = 13


def _build():
    mesh = plsc.VectorSubcoreMesh(core_axis_name="c", subcore_axis_name="s")

    @functools.partial(
        pl.kernel,
        out_type=jax.ShapeDtypeStruct((U, NG, 16), jnp.float32),
        mesh=mesh,
        compiler_params=pltpu.CompilerParams(use_tc_tiling_on_sc=False),
        scratch_types=[
            pltpu.VMEM((CH, 2, IDXC), jnp.int32),
            pltpu.VMEM((CH, D), jnp.float32),
            pltpu.VMEM((CH, NG, 16), jnp.float32),
            pltpu.VMEM((NG * 16, D), jnp.float32),
            pltpu.VMEM((NG * 16, D), jnp.float32),
            pltpu.SemaphoreType.DMA,
            pltpu.SemaphoreType.DMA,
        ],
    )
    def score_kernel(uf_hbm, if_hbm, ti_hbm, out_hbm,
                     idx_c, uvec_c, scores_c, rows0, rows1, sem0, sem1):
        def lane_perm(v, idx):
            dnums = lax.GatherDimensionNumbers(
                offset_dims=(), collapsed_slice_dims=(0,),
                start_index_map=(0,))
            return lax.gather(
                v, idx[:, None], dnums, (1,),
                mode=lax.GatherScatterMode.PROMISE_IN_BOUNDS)

        wid = lax.axis_index("s") * 2 + lax.axis_index("c")
        base_user = wid * UPW
        lane = lax.iota(jnp.int32, 16)

        def start_gather(uu, rows_ref, sem):
            pltpu.async_copy(if_hbm.at[idx_c.at[uu, 0]],
                             rows_ref.at[pl.ds(0, IDXC)], sem)
            pltpu.async_copy(if_hbm.at[idx_c.at[uu, 1]],
                             rows_ref.at[pl.ds(IDXC, IDXC)], sem)

        def wait_gather(uu, rows_ref, sem):
            pltpu.make_async_copy(if_hbm.at[idx_c.at[uu, 0]],
                                  rows_ref.at[pl.ds(0, IDXC)], sem).wait()
            pltpu.make_async_copy(if_hbm.at[idx_c.at[uu, 1]],
                                  rows_ref.at[pl.ds(IDXC, IDXC)], sem).wait()

        def compute_user(uu, rows_ref):
            def group(g, _):
                u0 = uvec_c[uu, pl.ds(0, 16)]
                u1 = uvec_c[uu, pl.ds(16, 16)]
                u2 = uvec_c[uu, pl.ds(32, 16)]
                u3 = uvec_c[uu, pl.ds(48, 16)]
                acc = jnp.zeros((16,), jnp.float32)
                for j in range(16):
                    li = g * 16 + j
                    part = (rows_ref[li, pl.ds(0, 16)] * u0
                            + rows_ref[li, pl.ds(16, 16)] * u1
                            + rows_ref[li, pl.ds(32, 16)] * u2
                            + rows_ref[li, pl.ds(48, 16)] * u3)
                    for sh in (8, 4, 2, 1):
                        part = part + lane_perm(part, lane ^ sh)
                    acc = jnp.where(lane == j, part, acc)
                scores_c[uu, g, :] = acc
                return 0

            lax.fori_loop(0, NG, group, 0)

        def chunk_body(ci, _):
            cb = base_user + ci * CH
            pltpu.sync_copy(ti_hbm.at[pl.ds(cb, CH)], idx_c)
            pltpu.sync_copy(uf_hbm.at[pl.ds(cb, CH)], uvec_c)
            start_gather(0, rows0, sem0)

            def pair(p, _):
                start_gather(2 * p + 1, rows1, sem1)
                wait_gather(2 * p, rows0, sem0)
                compute_user(2 * p, rows0)

                @pl.when(p < CH // 2 - 1)
                def _prefetch():
                    start_gather(2 * p + 2, rows0, sem0)

                wait_gather(2 * p + 1, rows1, sem1)
                compute_user(2 * p + 1, rows1)
                return 0

            lax.fori_loop(0, CH // 2, pair, 0)
            pltpu.sync_copy(scores_c, out_hbm.at[pl.ds(cb, CH)])
            return 0

        lax.fori_loop(0, UPW // CH, chunk_body, 0)

    return score_kernel


_score_kernel = _build()


def kernel(user_factors, item_factors, test_items):
    ti3 = test_items.astype(jnp.int32).reshape(U, 2, IDXC)
    out = _score_kernel(user_factors, item_factors, ti3)
    return out.reshape(U, NG * 16)[:, :L]


# speedup vs baseline: 2.2009x; 1.2136x over previous
import functools

import jax
import jax.numpy as jnp
from jax import lax
from jax.experimental import pallas as pl
from jax.experimental.pallas import tpu as pltpu
from jax.experimental.pallas import tpu_sc as plsc

U = 16384
N_ITEMS = 1000000
D = 64
L = 200
LP = 208

NW = 32
UPW = U // NW
CH = 32
NCH = UPW // CH
IDXA = 104
IDXB = 96
NG = LP // 16


def _build():
    mesh = plsc.VectorSubcoreMesh(core_axis_name="c", subcore_axis_name="s")

    @functools.partial(
        pl.kernel,
        out_type=jax.ShapeDtypeStruct((U * LP,), jnp.float32),
        mesh=mesh,
        compiler_params=pltpu.CompilerParams(use_tc_tiling_on_sc=False),
        scratch_types=[
            pltpu.VMEM((2, CH, L), jnp.int32),
            pltpu.VMEM((2, CH * D), jnp.float32),
            pltpu.VMEM((2, CH * LP), jnp.float32),
            pltpu.VMEM((LP, D), jnp.float32),
            pltpu.VMEM((LP, D), jnp.float32),
            pltpu.VMEM((LP, D), jnp.float32),
            pltpu.VMEM((LP, D), jnp.float32),
            pltpu.VMEM((LP, D), jnp.float32),
            pltpu.VMEM((LP, D), jnp.float32),
            pltpu.SemaphoreType.DMA,
            pltpu.SemaphoreType.DMA,
            pltpu.SemaphoreType.DMA,
            pltpu.SemaphoreType.DMA,
            pltpu.SemaphoreType.DMA,
            pltpu.SemaphoreType.DMA,
            pltpu.SemaphoreType.DMA,
            pltpu.SemaphoreType.DMA,
            pltpu.SemaphoreType.DMA,
            pltpu.SemaphoreType.DMA,
        ],
    )
    def score_kernel(uf_hbm, if_hbm, ti_hbm, out_hbm,
                     idx_c, uvec_c, scores_c,
                     rows0, rows1, rows2, rows3, rows4, rows5,
                     sr0, sr1, sr2, sr3, sr4, sr5, sg0, sg1, so0, so1):
        rbufs = (rows0, rows1, rows2, rows3, rows4, rows5)
        srow = (sr0, sr1, sr2, sr3, sr4, sr5)
        sstage = (sg0, sg1)
        sout = (so0, so1)

        def lane_perm(v, idx):
            dnums = lax.GatherDimensionNumbers(
                offset_dims=(), collapsed_slice_dims=(0,),
                start_index_map=(0,))
            return lax.gather(
                v, idx[:, None], dnums, (1,),
                mode=lax.GatherScatterMode.PROMISE_IN_BOUNDS)

        wid = lax.axis_index("s") * 2 + lax.axis_index("c")
        base_user = wid * UPW
        lane = lax.iota(jnp.int32, 16)

        def stage_descs(ci, sl):
            cb = base_user + ci * CH
            return (
                pltpu.make_async_copy(ti_hbm.at[pl.ds(cb, CH)],
                                      idx_c.at[sl], sstage[sl]),
                pltpu.make_async_copy(uf_hbm.at[pl.ds(cb * D, CH * D)],
                                      uvec_c.at[sl], sstage[sl]),
            )

        def stage_start(ci, sl):
            for c in stage_descs(ci, sl):
                c.start()

        def stage_wait(ci, sl):
            for c in stage_descs(ci, sl):
                c.wait()

        def gather_descs(sl, uu, rows_ref, sem):
            return (
                pltpu.make_async_copy(
                    if_hbm.at[idx_c.at[sl, uu, pl.ds(0, IDXA)]],
                    rows_ref.at[pl.ds(0, IDXA)], sem),
                pltpu.make_async_copy(
                    if_hbm.at[idx_c.at[sl, uu, pl.ds(IDXA, IDXB)]],
                    rows_ref.at[pl.ds(IDXA, IDXB)], sem),
            )

        def gather_start(sl, uu, rows_ref, sem):
            for c in gather_descs(sl, uu, rows_ref, sem):
                c.start()

        def gather_wait(sl, uu, rows_ref, sem):
            for c in gather_descs(sl, uu, rows_ref, sem):
                c.wait()

        def out_descs(ci, sl):
            cb = base_user + ci * CH
            return pltpu.make_async_copy(
                scores_c.at[sl], out_hbm.at[pl.ds(cb * LP, CH * LP)],
                sout[sl])

        def compute_user(sl, uu, rows_ref):
            def group(g, _):
                ud = pl.multiple_of(uu * D, 16)
                u0 = uvec_c[sl, pl.ds(ud, 16)]
                u1 = uvec_c[sl, pl.ds(ud + 16, 16)]
                u2 = uvec_c[sl, pl.ds(ud + 32, 16)]
                u3 = uvec_c[sl, pl.ds(ud + 48, 16)]
                vs = []
                for j in range(16):
                    li = g * 16 + j
                    vs.append(rows_ref[li, pl.ds(0, 16)] * u0
                              + rows_ref[li, pl.ds(16, 16)] * u1
                              + rows_ref[li, pl.ds(32, 16)] * u2
                              + rows_ref[li, pl.ds(48, 16)] * u3)
                for dist in (8, 4, 2, 1):
                    half = len(vs) // 2
                    m = (lane & dist) == 0
                    vs = [jnp.where(m,
                                    vs[k] + lane_perm(vs[k], lane ^ dist),
                                    vs[k + half]
                                    + lane_perm(vs[k + half], lane ^ dist))
                          for k in range(half)]
                base_o = pl.multiple_of(uu * LP + g * 16, 16)
                scores_c[sl, pl.ds(base_o, 16)] = vs[0]
                return 0

            lax.fori_loop(0, NG, group, 0)

        def run_chunk(ci, sl, c2, first, last):
            stage_wait(ci, sl)

            @pl.when(jnp.logical_not(last))
            def _prefetch_stage():
                stage_start(ci + 1, 1 - sl)

            @pl.when(jnp.logical_not(first))
            def _drain_out():
                out_descs(ci - 2, sl).wait()

            for w in range(5):
                gather_start(sl, w, rbufs[w], srow[w])

            def sext(q, _):
                for b in range(6):
                    u = 6 * q + b

                    @pl.when(u + 5 < CH)
                    def _prefetch_rows():
                        gather_start(sl, u + 5, rbufs[(b + 5) % 6],
                                     srow[(b + 5) % 6])

                    gather_wait(sl, u, rbufs[b], srow[b])
                    compute_user(sl, u, rbufs[b])
                return 0

            lax.fori_loop(0, CH // 6, sext, 0)
            for u in (30, 31):
                gather_wait(sl, u, rbufs[u % 6], srow[u % 6])
                compute_user(sl, u, rbufs[u % 6])
            out_descs(ci, sl).start()

        stage_start(0, 0)

        def chunk_pair(c2, _):
            run_chunk(2 * c2, 0, c2, c2 == 0, jnp.bool_(False))
            run_chunk(2 * c2 + 1, 1, c2, c2 == 0, c2 == NCH // 2 - 1)
            return 0

        lax.fori_loop(0, NCH // 2, chunk_pair, 0)
        out_descs(NCH - 2, 0).wait()
        out_descs(NCH - 1, 1).wait()

    return score_kernel


_score_kernel = _build()


def kernel(user_factors, item_factors, test_items):
    out = _score_kernel(user_factors.reshape(U * D), item_factors,
                        test_items.astype(jnp.int32))
    return out.reshape(U, LP)[:, :L]


# speedup vs baseline: 2.2708x; 1.0318x over previous
import functools

import jax
import jax.numpy as jnp
from jax import lax
from jax.experimental import pallas as pl
from jax.experimental.pallas import tpu as pltpu
from jax.experimental.pallas import tpu_sc as plsc

U = 16384
N_ITEMS = 1000000
D = 64
L = 200
LP = 208

NW = 32
UPW = U // NW
CH = 64
NCH = UPW // CH
IDXA = 104
IDXB = 96
NG = LP // 16


def _build():
    mesh = plsc.VectorSubcoreMesh(core_axis_name="c", subcore_axis_name="s")

    @functools.partial(
        pl.kernel,
        out_type=jax.ShapeDtypeStruct((U * LP,), jnp.float32),
        mesh=mesh,
        compiler_params=pltpu.CompilerParams(use_tc_tiling_on_sc=False),
        scratch_types=[
            pltpu.VMEM((2, CH, L), jnp.int32),
            pltpu.VMEM((2, CH * D), jnp.float32),
            pltpu.VMEM((2, CH * LP), jnp.float32),
            pltpu.VMEM((LP, D), jnp.float32),
            pltpu.VMEM((LP, D), jnp.float32),
            pltpu.VMEM((LP, D), jnp.float32),
            pltpu.VMEM((LP, D), jnp.float32),
            pltpu.SemaphoreType.DMA,
            pltpu.SemaphoreType.DMA,
            pltpu.SemaphoreType.DMA,
            pltpu.SemaphoreType.DMA,
            pltpu.SemaphoreType.DMA,
            pltpu.SemaphoreType.DMA,
            pltpu.SemaphoreType.DMA,
            pltpu.SemaphoreType.DMA,
        ],
    )
    def score_kernel(uf_hbm, if_hbm, ti_hbm, out_hbm,
                     idx_c, uvec_c, scores_c, rows0, rows1, rows2, rows3,
                     sr0, sr1, sr2, sr3, sg0, sg1, so0, so1):
        rbufs = (rows0, rows1, rows2, rows3)
        srow = (sr0, sr1, sr2, sr3)
        sstage = (sg0, sg1)
        sout = (so0, so1)

        def lane_perm(v, idx):
            dnums = lax.GatherDimensionNumbers(
                offset_dims=(), collapsed_slice_dims=(0,),
                start_index_map=(0,))
            return lax.gather(
                v, idx[:, None], dnums, (1,),
                mode=lax.GatherScatterMode.PROMISE_IN_BOUNDS)

        wid = lax.axis_index("s") * 2 + lax.axis_index("c")
        base_user = wid * UPW
        lane = lax.iota(jnp.int32, 16)

        def stage_descs(ci, sl):
            cb = base_user + ci * CH
            return (
                pltpu.make_async_copy(ti_hbm.at[pl.ds(cb, CH)],
                                      idx_c.at[sl], sstage[sl]),
                pltpu.make_async_copy(uf_hbm.at[pl.ds(cb * D, CH * D)],
                                      uvec_c.at[sl], sstage[sl]),
            )

        def stage_start(ci, sl):
            for c in stage_descs(ci, sl):
                c.start()

        def stage_wait(ci, sl):
            for c in stage_descs(ci, sl):
                c.wait()

        def gather_descs(sl, uu, rows_ref, sem):
            return (
                pltpu.make_async_copy(
                    if_hbm.at[idx_c.at[sl, uu, pl.ds(0, IDXA)]],
                    rows_ref.at[pl.ds(0, IDXA)], sem),
                pltpu.make_async_copy(
                    if_hbm.at[idx_c.at[sl, uu, pl.ds(IDXA, IDXB)]],
                    rows_ref.at[pl.ds(IDXA, IDXB)], sem),
            )

        def gather_start(sl, uu, rows_ref, sem):
            for c in gather_descs(sl, uu, rows_ref, sem):
                c.start()

        def gather_wait(sl, uu, rows_ref, sem):
            for c in gather_descs(sl, uu, rows_ref, sem):
                c.wait()

        def out_descs(ci, sl):
            cb = base_user + ci * CH
            return pltpu.make_async_copy(
                scores_c.at[sl], out_hbm.at[pl.ds(cb * LP, CH * LP)],
                sout[sl])

        def compute_user(sl, uu, rows_ref):
            def group(g, _):
                ud = pl.multiple_of(uu * D, 16)
                u0 = uvec_c[sl, pl.ds(ud, 16)]
                u1 = uvec_c[sl, pl.ds(ud + 16, 16)]
                u2 = uvec_c[sl, pl.ds(ud + 32, 16)]
                u3 = uvec_c[sl, pl.ds(ud + 48, 16)]
                vs = []
                for j in range(16):
                    li = g * 16 + j
                    vs.append(rows_ref[li, pl.ds(0, 16)] * u0
                              + rows_ref[li, pl.ds(16, 16)] * u1
                              + rows_ref[li, pl.ds(32, 16)] * u2
                              + rows_ref[li, pl.ds(48, 16)] * u3)
                for dist in (8, 4, 2, 1):
                    half = len(vs) // 2
                    m = (lane & dist) == 0
                    vs = [jnp.where(m,
                                    vs[k] + lane_perm(vs[k], lane ^ dist),
                                    vs[k + half]
                                    + lane_perm(vs[k + half], lane ^ dist))
                          for k in range(half)]
                base_o = pl.multiple_of(uu * LP + g * 16, 16)
                scores_c[sl, pl.ds(base_o, 16)] = vs[0]
                return 0

            lax.fori_loop(0, NG, group, 0)

        def run_chunk(ci, sl, c2, first, last):
            stage_wait(ci, sl)

            @pl.when(jnp.logical_not(last))
            def _prefetch_stage():
                stage_start(ci + 1, 1 - sl)

            @pl.when(jnp.logical_not(first))
            def _drain_out():
                out_descs(ci - 2, sl).wait()

            gather_start(sl, 0, rbufs[0], srow[0])
            gather_start(sl, 1, rbufs[1], srow[1])
            gather_start(sl, 2, rbufs[2], srow[2])

            def quad(q, _):
                for b in range(4):
                    u = 4 * q + b

                    @pl.when(u + 3 < CH)
                    def _prefetch_rows():
                        gather_start(sl, u + 3, rbufs[(b + 3) % 4],
                                     srow[(b + 3) % 4])

                    gather_wait(sl, u, rbufs[b], srow[b])
                    compute_user(sl, u, rbufs[b])
                return 0

            lax.fori_loop(0, CH // 4, quad, 0)
            out_descs(ci, sl).start()

        stage_start(0, 0)

        def chunk_pair(c2, _):
            run_chunk(2 * c2, 0, c2, c2 == 0, jnp.bool_(False))
            run_chunk(2 * c2 + 1, 1, c2, c2 == 0, c2 == NCH // 2 - 1)
            return 0

        lax.fori_loop(0, NCH // 2, chunk_pair, 0)
        out_descs(NCH - 2, 0).wait()
        out_descs(NCH - 1, 1).wait()

    return score_kernel


_score_kernel = _build()


def kernel(user_factors, item_factors, test_items):
    out = _score_kernel(user_factors.reshape(U * D), item_factors,
                        test_items.astype(jnp.int32))
    return out.reshape(U, LP)[:, :L]
